# baseline (speedup 1.0000x reference)
import functools

import jax
import jax.numpy as jnp
from jax import lax
from jax.experimental import pallas as pl
from jax.experimental.pallas import tpu as pltpu
from jax.experimental.pallas import tpu_sc as plsc

NC = 2
NS = 16
LANES = 16
N_OUT = 2048
SG = 24


@functools.lru_cache(maxsize=None)
def _build(B, EC, D):
    NBG = EC // LANES
    rpt = N_OUT // (NC * NS)
    nvec = D // LANES

    mesh = plsc.VectorSubcoreMesh(
        core_axis_name="c", subcore_axis_name="s",
        num_cores=NC, num_subcores=NS)

    @functools.partial(
        pl.kernel,
        out_type=jax.ShapeDtypeStruct((B, N_OUT, D), jnp.float32),
        mesh=mesh,
        scratch_types=[
            pltpu.VMEM((EC + 2 * SG,), jnp.int32),
            pltpu.VMEM((EC + 2 * SG,), jnp.int32),
            pltpu.VMEM((EC + 2 * SG,), jnp.float32),
            pltpu.VMEM((SG, D), jnp.float32),
            pltpu.VMEM((SG, D), jnp.float32),
            pltpu.VMEM((rpt, D), jnp.float32),
            pltpu.SemaphoreType.DMA,
            pltpu.SemaphoreType.DMA,
            pltpu.SemaphoreType.DMA,
            pltpu.SemaphoreType.DMA,
        ],
        compiler_params=pltpu.CompilerParams(needs_layout_passes=False),
    )
    def run(score_h, idx_h, disp_h, w_h, out_h,
            ia_v, da_v, sa_v, gbuf0, gbuf1, acc, sem_in, sem0, sem1, sem_wb):
        c = lax.axis_index("c")
        s = lax.axis_index("s")
        row0 = (c * NS + s) * rpt
        zero = jnp.zeros((LANES,), jnp.float32)
        gbufs = (gbuf0, gbuf1)
        sems = (sem0, sem1)

        def fetch(gg, buf, sem):
            pltpu.async_copy(w_h.at[ia_v.at[pl.ds(gg * SG, SG)]], buf, sem)

        def drain(buf, sem):
            pltpu.make_async_copy(w_h.at[pl.ds(0, SG)], buf, sem).wait()

        def accumulate(k0, buf):
            K = 16

            def row(r, carry2):
                lr = da_v[pl.ds(k0 + r, LANES)][0]
                sv = jnp.broadcast_to(sa_v[pl.ds(k0 + r, LANES)][0], (LANES,))
                for v0 in range(0, nvec, K):
                    prods = [buf[r, pl.ds(v * LANES, LANES)] * sv
                             for v in range(v0, v0 + K)]
                    for i, v in enumerate(range(v0, v0 + K)):
                        plsc.addupdate(acc.at[lr, pl.ds(v * LANES, LANES)],
                                       prods[i])
                return carry2
            lax.fori_loop(0, SG, row, 0)

        wb_prev = []
        for b in range(B):
            cp_i = pltpu.async_copy(idx_h.at[b], ia_v.at[pl.ds(0, EC)], sem_in)
            cp_d = pltpu.async_copy(disp_h.at[b], da_v.at[pl.ds(0, EC)], sem_in)
            cp_s = pltpu.async_copy(score_h.at[b], sa_v.at[pl.ds(0, EC)], sem_in)

            def zrow(r, carry):
                for v in range(nvec):
                    acc[r, pl.ds(v * LANES, LANES)] = zero
                return carry

            cp_i.wait()
            cp_d.wait()
            cp_s.wait()

            def scan(g, cur):
                sl = pl.ds(g * LANES, LANES)
                iv = ia_v[sl]
                sv = sa_v[sl]
                lr = da_v[sl] - row0
                m = (lr >= 0) & (lr < rpt)
                plsc.store_compressed(da_v.at[pl.ds(cur, LANES)], lr, mask=m)
                plsc.store_compressed(ia_v.at[pl.ds(cur, LANES)], iv, mask=m)
                plsc.store_compressed(sa_v.at[pl.ds(cur, LANES)], sv, mask=m)
                return cur + plsc.all_reduce_population_count(m)[0]

            with jax.named_scope("ph_scan"):
                m_cnt = lax.fori_loop(0, NBG, scan, jnp.int32(0))

            if wb_prev:
                wb_prev.pop().wait()
            with jax.named_scope("ph_zero"):
                lax.fori_loop(0, rpt, zrow, 0)

            for q in range(2):
                padq = pl.ds(m_cnt + q * LANES, LANES)
                da_v[padq] = jnp.full((LANES,), rpt - 1, jnp.int32)
                ia_v[padq] = jnp.zeros((LANES,), jnp.int32)
                sa_v[padq] = zero

            ng = (m_cnt + SG - 1) // SG
            pl.when(ng > 0)(lambda: fetch(0, gbuf0, sem0))

            def group(gg, carry):
                for p in range(2):
                    @pl.when(lax.rem(gg, 2) == p)
                    def _():
                        pl.when(gg + 1 < ng)(
                            lambda: fetch(gg + 1, gbufs[1 - p], sems[1 - p]))
                        with jax.named_scope("ph_drain"):
                            drain(gbufs[p], sems[p])
                        with jax.named_scope("ph_comp"):
                            accumulate(gg * SG, gbufs[p])
                return carry

            lax.fori_loop(0, ng, group, 0)
            wb_prev.append(pltpu.async_copy(
                acc.at[pl.ds(0, rpt)], out_h.at[b, pl.ds(row0, rpt)], sem_wb))
        wb_prev.pop().wait()

    return run


def kernel(score, indices, dispatch, n, weight):
    B, E, C = score.shape
    V, D = weight.shape
    EC = E * C
    score2 = score.reshape(B, EC)
    idx2 = indices.reshape(B, EC)
    disp2 = jnp.minimum(dispatch.reshape(B, EC), n - 1).astype(jnp.int32)
    return _build(B, EC, D)(score2, idx2, disp2, weight)


# speedup vs baseline: 1.1555x; 1.1555x over previous
import functools

import jax
import jax.numpy as jnp
from jax import lax
from jax.experimental import pallas as pl
from jax.experimental.pallas import tpu as pltpu
from jax.experimental.pallas import tpu_sc as plsc

NC = 2
NS = 16
LANES = 16
N_OUT = 2048
SG = 16


@functools.lru_cache(maxsize=None)
def _build(B, EC, D):
    NBG = EC // LANES
    rpt = N_OUT // (NC * NS)
    nvec = D // LANES

    mesh = plsc.VectorSubcoreMesh(
        core_axis_name="c", subcore_axis_name="s",
        num_cores=NC, num_subcores=NS)

    @functools.partial(
        pl.kernel,
        out_type=jax.ShapeDtypeStruct((B, N_OUT, D), jnp.float32),
        mesh=mesh,
        scratch_types=[
            pltpu.VMEM((EC + 2 * SG,), jnp.int32),
            pltpu.VMEM((EC + 2 * SG,), jnp.int32),
            pltpu.VMEM((EC + 2 * SG,), jnp.float32),
            pltpu.VMEM((SG, D), jnp.float32),
            pltpu.VMEM((SG, D), jnp.float32),
            pltpu.VMEM((SG, D), jnp.float32),
            pltpu.VMEM((rpt, D), jnp.float32),
            pltpu.SemaphoreType.DMA,
            pltpu.SemaphoreType.DMA,
            pltpu.SemaphoreType.DMA,
            pltpu.SemaphoreType.DMA,
            pltpu.SemaphoreType.DMA,
        ],
        compiler_params=pltpu.CompilerParams(needs_layout_passes=False),
    )
    def run(score_h, idx_h, disp_h, w_h, out_h,
            ia_v, da_v, sa_v, gbuf0, gbuf1, gbuf2, acc,
            sem_in, sem0, sem1, sem2, sem_wb):
        c = lax.axis_index("c")
        s = lax.axis_index("s")
        row0 = (c * NS + s) * rpt
        zero = jnp.zeros((LANES,), jnp.float32)
        gbufs = (gbuf0, gbuf1, gbuf2)
        sems = (sem0, sem1, sem2)
        NBUF = len(gbufs)

        def fetch(gg, buf, sem):
            pltpu.async_copy(w_h.at[ia_v.at[pl.ds(gg * SG, SG)]], buf, sem)

        def drain(buf, sem):
            pltpu.make_async_copy(w_h.at[pl.ds(0, SG)], buf, sem).wait()

        def accumulate(k0, buf):
            K = 16

            def row(r, carry2):
                lr = da_v[pl.ds(k0 + r, LANES)][0]
                sv = jnp.broadcast_to(sa_v[pl.ds(k0 + r, LANES)][0], (LANES,))
                for v0 in range(0, nvec, K):
                    prods = [buf[r, pl.ds(v * LANES, LANES)] * sv
                             for v in range(v0, v0 + K)]
                    for i, v in enumerate(range(v0, v0 + K)):
                        plsc.addupdate(acc.at[lr, pl.ds(v * LANES, LANES)],
                                       prods[i])
                return carry2
            lax.fori_loop(0, SG, row, 0)

        wb_prev = []
        for b in range(B):
            cp_i = pltpu.async_copy(idx_h.at[b], ia_v.at[pl.ds(0, EC)], sem_in)
            cp_d = pltpu.async_copy(disp_h.at[b], da_v.at[pl.ds(0, EC)], sem_in)
            cp_s = pltpu.async_copy(score_h.at[b], sa_v.at[pl.ds(0, EC)], sem_in)

            def zrow(r, carry):
                for v in range(nvec):
                    acc[r, pl.ds(v * LANES, LANES)] = zero
                return carry

            cp_i.wait()
            cp_d.wait()
            cp_s.wait()

            def scan(g, cur):
                sl = pl.ds(g * LANES, LANES)
                iv = ia_v[sl]
                sv = sa_v[sl]
                lr = da_v[sl] - row0
                m = (lr >= 0) & (lr < rpt)
                plsc.store_compressed(da_v.at[pl.ds(cur, LANES)], lr, mask=m)
                plsc.store_compressed(ia_v.at[pl.ds(cur, LANES)], iv, mask=m)
                plsc.store_compressed(sa_v.at[pl.ds(cur, LANES)], sv, mask=m)
                return cur + plsc.all_reduce_population_count(m)[0]

            with jax.named_scope("ph_scan"):
                m_cnt = lax.fori_loop(0, NBG, scan, jnp.int32(0))

            if wb_prev:
                wb_prev.pop().wait()
            with jax.named_scope("ph_zero"):
                lax.fori_loop(0, rpt, zrow, 0)

            for q in range(2):
                padq = pl.ds(m_cnt + q * LANES, LANES)
                da_v[padq] = jnp.full((LANES,), rpt - 1, jnp.int32)
                ia_v[padq] = jnp.zeros((LANES,), jnp.int32)
                sa_v[padq] = zero

            ng = (m_cnt + SG - 1) // SG
            pl.when(ng > 0)(lambda: fetch(0, gbuf0, sem0))
            pl.when(ng > 1)(lambda: fetch(1, gbuf1, sem1))

            def group(gg, carry):
                for p in range(NBUF):
                    @pl.when(lax.rem(gg, NBUF) == p)
                    def _():
                        q = (p + 2) % NBUF
                        pl.when(gg + 2 < ng)(
                            lambda: fetch(gg + 2, gbufs[q], sems[q]))
                        with jax.named_scope("ph_drain"):
                            drain(gbufs[p], sems[p])
                        with jax.named_scope("ph_comp"):
                            accumulate(gg * SG, gbufs[p])
                return carry

            lax.fori_loop(0, ng, group, 0)
            wb_prev.append(pltpu.async_copy(
                acc.at[pl.ds(0, rpt)], out_h.at[b, pl.ds(row0, rpt)], sem_wb))
        wb_prev.pop().wait()

    return run


def kernel(score, indices, dispatch, n, weight):
    B, E, C = score.shape
    V, D = weight.shape
    EC = E * C
    score2 = score.reshape(B, EC)
    idx2 = indices.reshape(B, EC)
    disp2 = jnp.minimum(dispatch.reshape(B, EC), n - 1).astype(jnp.int32)
    return _build(B, EC, D)(score2, idx2, disp2, weight)


# speedup vs baseline: 1.2467x; 1.0789x over previous
import functools

import jax
import jax.numpy as jnp
from jax import lax
from jax.experimental import pallas as pl
from jax.experimental.pallas import tpu as pltpu
from jax.experimental.pallas import tpu_sc as plsc

NC = 2
NS = 16
LANES = 16
N_OUT = 2048
SG = 8


@functools.lru_cache(maxsize=None)
def _build(B, EC, D):
    NBG = EC // LANES
    rpt = N_OUT // (NC * NS)
    nvec = D // LANES

    mesh = plsc.VectorSubcoreMesh(
        core_axis_name="c", subcore_axis_name="s",
        num_cores=NC, num_subcores=NS)

    @functools.partial(
        pl.kernel,
        out_type=jax.ShapeDtypeStruct((B, N_OUT, D), jnp.float32),
        mesh=mesh,
        scratch_types=[
            pltpu.VMEM((EC + 2 * LANES,), jnp.int32),
            pltpu.VMEM((EC + 2 * LANES,), jnp.int32),
            pltpu.VMEM((EC + 2 * LANES,), jnp.float32),
            pltpu.VMEM((SG, D), jnp.float32),
            pltpu.VMEM((SG, D), jnp.float32),
            pltpu.VMEM((SG, D), jnp.float32),
            pltpu.VMEM((SG, D), jnp.float32),
            pltpu.VMEM((SG, D), jnp.float32),
            pltpu.VMEM((SG, D), jnp.float32),
            pltpu.VMEM((rpt, D), jnp.float32),
            pltpu.SemaphoreType.DMA,
            pltpu.SemaphoreType.DMA,
            pltpu.SemaphoreType.DMA,
            pltpu.SemaphoreType.DMA,
            pltpu.SemaphoreType.DMA,
            pltpu.SemaphoreType.DMA,
            pltpu.SemaphoreType.DMA,
            pltpu.SemaphoreType.DMA,
        ],
        compiler_params=pltpu.CompilerParams(needs_layout_passes=False),
    )
    def run(score_h, idx_h, disp_h, w_h, out_h,
            ia_v, da_v, sa_v, gbuf0, gbuf1, gbuf2, gbuf3, gbuf4, gbuf5, acc,
            sem_in, sem0, sem1, sem2, sem3, sem4, sem5, sem_wb):
        c = lax.axis_index("c")
        s = lax.axis_index("s")
        row0 = (c * NS + s) * rpt
        zero = jnp.zeros((LANES,), jnp.float32)
        gbufs = (gbuf0, gbuf1, gbuf2, gbuf3, gbuf4, gbuf5)
        sems = (sem0, sem1, sem2, sem3, sem4, sem5)
        NBUF = len(gbufs)

        def fetch(gg, buf, sem):
            pltpu.async_copy(w_h.at[ia_v.at[pl.ds(gg * SG, SG)]], buf, sem)

        def drain(buf, sem):
            pltpu.make_async_copy(w_h.at[pl.ds(0, SG)], buf, sem).wait()

        def accumulate(k0, buf):
            K = 16

            def row(r, carry2):
                lr = da_v[pl.ds(k0 + r, LANES)][0]
                sv = jnp.broadcast_to(sa_v[pl.ds(k0 + r, LANES)][0], (LANES,))
                for v0 in range(0, nvec, K):
                    prods = [buf[r, pl.ds(v * LANES, LANES)] * sv
                             for v in range(v0, v0 + K)]
                    for i, v in enumerate(range(v0, v0 + K)):
                        plsc.addupdate(acc.at[lr, pl.ds(v * LANES, LANES)],
                                       prods[i])
                return carry2
            lax.fori_loop(0, SG, row, 0)

        wb_prev = []
        for b in range(B):
            cp_i = pltpu.async_copy(idx_h.at[b], ia_v.at[pl.ds(0, EC)], sem_in)
            cp_d = pltpu.async_copy(disp_h.at[b], da_v.at[pl.ds(0, EC)], sem_in)
            cp_s = pltpu.async_copy(score_h.at[b], sa_v.at[pl.ds(0, EC)], sem_in)

            def zrow(r, carry):
                for v in range(nvec):
                    acc[r, pl.ds(v * LANES, LANES)] = zero
                return carry

            cp_i.wait()
            cp_d.wait()
            cp_s.wait()

            def scan(g, cur):
                sl = pl.ds(g * LANES, LANES)
                iv = ia_v[sl]
                sv = sa_v[sl]
                lr = da_v[sl] - row0
                m = (lr >= 0) & (lr < rpt)
                plsc.store_compressed(da_v.at[pl.ds(cur, LANES)], lr, mask=m)
                plsc.store_compressed(ia_v.at[pl.ds(cur, LANES)], iv, mask=m)
                plsc.store_compressed(sa_v.at[pl.ds(cur, LANES)], sv, mask=m)
                return cur + plsc.all_reduce_population_count(m)[0]

            with jax.named_scope("ph_scan"):
                m_cnt = lax.fori_loop(0, NBG, scan, jnp.int32(0))

            if wb_prev:
                wb_prev.pop().wait()
            with jax.named_scope("ph_zero"):
                lax.fori_loop(0, rpt, zrow, 0)

            for q in range(2):
                padq = pl.ds(m_cnt + q * LANES, LANES)
                da_v[padq] = jnp.full((LANES,), rpt - 1, jnp.int32)
                ia_v[padq] = jnp.zeros((LANES,), jnp.int32)
                sa_v[padq] = zero

            ng = (m_cnt + SG - 1) // SG
            pl.when(ng > 0)(lambda: fetch(0, gbuf0, sem0))
            pl.when(ng > 1)(lambda: fetch(1, gbuf1, sem1))
            pl.when(ng > 2)(lambda: fetch(2, gbuf2, sem2))
            pl.when(ng > 3)(lambda: fetch(3, gbuf3, sem3))
            pl.when(ng > 4)(lambda: fetch(4, gbuf4, sem4))

            def group(gg, carry):
                for p in range(NBUF):
                    @pl.when(lax.rem(gg, NBUF) == p)
                    def _():
                        q = (p + 5) % NBUF
                        pl.when(gg + 5 < ng)(
                            lambda: fetch(gg + 5, gbufs[q], sems[q]))
                        with jax.named_scope("ph_drain"):
                            drain(gbufs[p], sems[p])
                        with jax.named_scope("ph_comp"):
                            accumulate(gg * SG, gbufs[p])
                return carry

            lax.fori_loop(0, ng, group, 0)
            wb_prev.append(pltpu.async_copy(
                acc.at[pl.ds(0, rpt)], out_h.at[b, pl.ds(row0, rpt)], sem_wb))
        wb_prev.pop().wait()

    return run


def kernel(score, indices, dispatch, n, weight):
    B, E, C = score.shape
    V, D = weight.shape
    EC = E * C
    score2 = score.reshape(B, EC)
    idx2 = indices.reshape(B, EC)
    disp2 = jnp.minimum(dispatch.reshape(B, EC), n - 1).astype(jnp.int32)
    return _build(B, EC, D)(score2, idx2, disp2, weight)


# speedup vs baseline: 1.2525x; 1.0047x over previous
import functools

import jax
import jax.numpy as jnp
from jax import lax
from jax.experimental import pallas as pl
from jax.experimental.pallas import tpu as pltpu
from jax.experimental.pallas import tpu_sc as plsc

NC = 2
NS = 16
LANES = 16
N_OUT = 2048
SG = 8


@functools.lru_cache(maxsize=None)
def _build(B, EC, D):
    NBG = EC // LANES
    rpt = N_OUT // (NC * NS)
    nvec = D // LANES

    mesh = plsc.VectorSubcoreMesh(
        core_axis_name="c", subcore_axis_name="s",
        num_cores=NC, num_subcores=NS)

    @functools.partial(
        pl.kernel,
        out_type=jax.ShapeDtypeStruct((B, N_OUT, D), jnp.float32),
        mesh=mesh,
        scratch_types=[
            pltpu.VMEM((EC + 2 * LANES,), jnp.int32),
            pltpu.VMEM((EC + 2 * LANES,), jnp.int32),
            pltpu.VMEM((EC + 2 * LANES,), jnp.float32),
            pltpu.VMEM((SG, D), jnp.float32),
            pltpu.VMEM((SG, D), jnp.float32),
            pltpu.VMEM((SG, D), jnp.float32),
            pltpu.VMEM((SG, D), jnp.float32),
            pltpu.VMEM((SG, D), jnp.float32),
            pltpu.VMEM((SG, D), jnp.float32),
            pltpu.VMEM((rpt, D), jnp.float32),
            pltpu.SemaphoreType.DMA,
            pltpu.SemaphoreType.DMA,
            pltpu.SemaphoreType.DMA,
            pltpu.SemaphoreType.DMA,
            pltpu.SemaphoreType.DMA,
            pltpu.SemaphoreType.DMA,
            pltpu.SemaphoreType.DMA,
            pltpu.SemaphoreType.DMA,
        ],
        compiler_params=pltpu.CompilerParams(needs_layout_passes=False),
    )
    def run(score_h, idx_h, disp_h, w_h, out_h,
            ia_v, da_v, sa_v, gbuf0, gbuf1, gbuf2, gbuf3, gbuf4, gbuf5, acc,
            sem_in, sem0, sem1, sem2, sem3, sem4, sem5, sem_wb):
        c = lax.axis_index("c")
        s = lax.axis_index("s")
        row0 = (c * NS + s) * rpt
        zero = jnp.zeros((LANES,), jnp.float32)
        gbufs = (gbuf0, gbuf1, gbuf2, gbuf3, gbuf4, gbuf5)
        sems = (sem0, sem1, sem2, sem3, sem4, sem5)
        NBUF = len(gbufs)

        def fetch(gg, buf, sem):
            pltpu.async_copy(w_h.at[ia_v.at[pl.ds(gg * SG, SG)]], buf, sem)

        def drain(buf, sem):
            pltpu.make_async_copy(w_h.at[pl.ds(0, SG)], buf, sem).wait()

        def accumulate(k0, buf):
            K = 16

            def row(r, carry2):
                lr = da_v[pl.ds(k0 + r, LANES)][0]
                sv = jnp.broadcast_to(sa_v[pl.ds(k0 + r, LANES)][0], (LANES,))
                for v0 in range(0, nvec, K):
                    prods = [buf[r, pl.ds(v * LANES, LANES)] * sv
                             for v in range(v0, v0 + K)]
                    for i, v in enumerate(range(v0, v0 + K)):
                        plsc.addupdate(acc.at[lr, pl.ds(v * LANES, LANES)],
                                       prods[i])
                return carry2
            lax.fori_loop(0, SG, row, 0)

        wb_prev = []
        for b in range(B):
            cp_i = pltpu.async_copy(idx_h.at[b], ia_v.at[pl.ds(0, EC)], sem_in)
            cp_d = pltpu.async_copy(disp_h.at[b], da_v.at[pl.ds(0, EC)], sem_in)
            cp_s = pltpu.async_copy(score_h.at[b], sa_v.at[pl.ds(0, EC)], sem_in)

            def zrow(r, carry):
                for v in range(nvec):
                    acc[r, pl.ds(v * LANES, LANES)] = zero
                return carry

            cp_i.wait()
            cp_d.wait()
            cp_s.wait()

            def scan(g, cur):
                sl = pl.ds(g * LANES, LANES)
                iv = ia_v[sl]
                sv = sa_v[sl]
                lr = da_v[sl] - row0
                m = (lr >= 0) & (lr < rpt)
                plsc.store_compressed(da_v.at[pl.ds(cur, LANES)], lr, mask=m)
                plsc.store_compressed(ia_v.at[pl.ds(cur, LANES)], iv, mask=m)
                plsc.store_compressed(sa_v.at[pl.ds(cur, LANES)], sv, mask=m)
                return cur + plsc.all_reduce_population_count(m)[0]

            m_cnt = lax.fori_loop(0, NBG, scan, jnp.int32(0))

            if wb_prev:
                wb_prev.pop().wait()
            lax.fori_loop(0, rpt, zrow, 0)

            for q in range(2):
                padq = pl.ds(m_cnt + q * LANES, LANES)
                da_v[padq] = jnp.full((LANES,), rpt - 1, jnp.int32)
                ia_v[padq] = jnp.zeros((LANES,), jnp.int32)
                sa_v[padq] = zero

            ng = (m_cnt + SG - 1) // SG
            pl.when(ng > 0)(lambda: fetch(0, gbuf0, sem0))
            pl.when(ng > 1)(lambda: fetch(1, gbuf1, sem1))
            pl.when(ng > 2)(lambda: fetch(2, gbuf2, sem2))
            pl.when(ng > 3)(lambda: fetch(3, gbuf3, sem3))
            pl.when(ng > 4)(lambda: fetch(4, gbuf4, sem4))

            def group(gg, carry):
                for p in range(NBUF):
                    @pl.when(lax.rem(gg, NBUF) == p)
                    def _():
                        q = (p + 5) % NBUF
                        pl.when(gg + 5 < ng)(
                            lambda: fetch(gg + 5, gbufs[q], sems[q]))
                        drain(gbufs[p], sems[p])
                        accumulate(gg * SG, gbufs[p])
                return carry

            lax.fori_loop(0, ng, group, 0)
            wb_prev.append(pltpu.async_copy(
                acc.at[pl.ds(0, rpt)], out_h.at[b, pl.ds(row0, rpt)], sem_wb))
        wb_prev.pop().wait()

    return run


def kernel(score, indices, dispatch, n, weight):
    B, E, C = score.shape
    V, D = weight.shape
    EC = E * C
    score2 = score.reshape(B, EC)
    idx2 = indices.reshape(B, EC)
    disp2 = jnp.minimum(dispatch.reshape(B, EC), n - 1).astype(jnp.int32)
    return _build(B, EC, D)(score2, idx2, disp2, weight)


# speedup vs baseline: 1.3152x; 1.0501x over previous
import functools

import jax
import jax.numpy as jnp
from jax import lax
from jax.experimental import pallas as pl
from jax.experimental.pallas import tpu as pltpu
from jax.experimental.pallas import tpu_sc as plsc

NC = 2
NS = 16
LANES = 16
N_OUT = 2048
SG = 8


@functools.lru_cache(maxsize=None)
def _build(B, EC, D):
    NBG = EC // LANES
    rpt = N_OUT // (NC * NS)
    nvec = D // LANES

    mesh = plsc.VectorSubcoreMesh(
        core_axis_name="c", subcore_axis_name="s",
        num_cores=NC, num_subcores=NS)

    @functools.partial(
        pl.kernel,
        out_type=jax.ShapeDtypeStruct((B, N_OUT, D), jnp.float32),
        mesh=mesh,
        scratch_types=[
            pltpu.VMEM((EC + 2 * LANES,), jnp.int32),
            pltpu.VMEM((EC + 2 * LANES,), jnp.int32),
            pltpu.VMEM((EC + 2 * LANES,), jnp.float32),
            pltpu.VMEM((SG, D), jnp.float32),
            pltpu.VMEM((SG, D), jnp.float32),
            pltpu.VMEM((SG, D), jnp.float32),
            pltpu.VMEM((SG, D), jnp.float32),
            pltpu.VMEM((SG, D), jnp.float32),
            pltpu.VMEM((SG, D), jnp.float32),
            pltpu.VMEM((rpt, D), jnp.float32),
            pltpu.SemaphoreType.DMA,
            pltpu.SemaphoreType.DMA,
            pltpu.SemaphoreType.DMA,
            pltpu.SemaphoreType.DMA,
            pltpu.SemaphoreType.DMA,
            pltpu.SemaphoreType.DMA,
            pltpu.SemaphoreType.DMA,
            pltpu.SemaphoreType.DMA,
        ],
        compiler_params=pltpu.CompilerParams(needs_layout_passes=False),
    )
    def run(score_h, idx_h, disp_h, w_h, out_h,
            ia_v, da_v, sa_v, gbuf0, gbuf1, gbuf2, gbuf3, gbuf4, gbuf5, acc,
            sem_in, sem0, sem1, sem2, sem3, sem4, sem5, sem_wb):
        c = lax.axis_index("c")
        s = lax.axis_index("s")
        row0 = (c * NS + s) * rpt
        zero = jnp.zeros((LANES,), jnp.float32)
        gbufs = (gbuf0, gbuf1, gbuf2, gbuf3, gbuf4, gbuf5)
        sems = (sem0, sem1, sem2, sem3, sem4, sem5)
        NBUF = len(gbufs)

        def fetch(gg, buf, sem):
            pltpu.async_copy(w_h.at[ia_v.at[pl.ds(gg * SG, SG)]], buf, sem)

        def drain(buf, sem):
            pltpu.make_async_copy(w_h.at[pl.ds(0, SG)], buf, sem).wait()

        def accumulate(k0, buf):
            K = 16

            def row(r, carry2):
                lr = da_v[pl.ds(k0 + r, LANES)][0]
                sv = jnp.broadcast_to(sa_v[pl.ds(k0 + r, LANES)][0], (LANES,))
                for v0 in range(0, nvec, K):
                    prods = [buf[r, pl.ds(v * LANES, LANES)] * sv
                             for v in range(v0, v0 + K)]
                    for i, v in enumerate(range(v0, v0 + K)):
                        plsc.addupdate(acc.at[lr, pl.ds(v * LANES, LANES)],
                                       prods[i])
                return carry2
            lax.fori_loop(0, SG, row, 0)

        wb_prev = []
        for b in range(B):
            cp_i = pltpu.async_copy(idx_h.at[b], ia_v.at[pl.ds(0, EC)], sem_in)
            cp_d = pltpu.async_copy(disp_h.at[b], da_v.at[pl.ds(0, EC)], sem_in)
            cp_s = pltpu.async_copy(score_h.at[b], sa_v.at[pl.ds(0, EC)], sem_in)

            def zrow(r, carry):
                for v in range(nvec):
                    acc[r, pl.ds(v * LANES, LANES)] = zero
                return carry

            cp_i.wait()
            cp_d.wait()
            cp_s.wait()

            def scan(g, cur):
                sl = pl.ds(g * LANES, LANES)
                iv = ia_v[sl]
                sv = sa_v[sl]
                lr = da_v[sl] - row0
                m = (lr >= 0) & (lr < rpt)
                plsc.store_compressed(da_v.at[pl.ds(cur, LANES)], lr, mask=m)
                plsc.store_compressed(ia_v.at[pl.ds(cur, LANES)], iv, mask=m)
                plsc.store_compressed(sa_v.at[pl.ds(cur, LANES)], sv, mask=m)
                return cur + plsc.all_reduce_population_count(m)[0]

            m_cnt = lax.fori_loop(0, NBG, scan, jnp.int32(0))

            for q in range(2):
                padq = pl.ds(m_cnt + q * LANES, LANES)
                da_v[padq] = jnp.full((LANES,), rpt - 1, jnp.int32)
                ia_v[padq] = jnp.zeros((LANES,), jnp.int32)
                sa_v[padq] = zero

            ng = (m_cnt + SG - 1) // SG
            pl.when(ng > 0)(lambda: fetch(0, gbuf0, sem0))
            pl.when(ng > 1)(lambda: fetch(1, gbuf1, sem1))
            pl.when(ng > 2)(lambda: fetch(2, gbuf2, sem2))
            pl.when(ng > 3)(lambda: fetch(3, gbuf3, sem3))
            pl.when(ng > 4)(lambda: fetch(4, gbuf4, sem4))

            if wb_prev:
                wb_prev.pop().wait()
            lax.fori_loop(0, rpt, zrow, 0)

            def group(gg, carry):
                for p in range(NBUF):
                    @pl.when(lax.rem(gg, NBUF) == p)
                    def _():
                        q = (p + 5) % NBUF
                        pl.when(gg + 5 < ng)(
                            lambda: fetch(gg + 5, gbufs[q], sems[q]))
                        drain(gbufs[p], sems[p])
                        accumulate(gg * SG, gbufs[p])
                return carry

            lax.fori_loop(0, ng, group, 0)
            wb_prev.append(pltpu.async_copy(
                acc.at[pl.ds(0, rpt)], out_h.at[b, pl.ds(row0, rpt)], sem_wb))
        wb_prev.pop().wait()

    return run


def kernel(score, indices, dispatch, n, weight):
    B, E, C = score.shape
    V, D = weight.shape
    EC = E * C
    score2 = score.reshape(B, EC)
    idx2 = indices.reshape(B, EC)
    disp2 = jnp.minimum(dispatch.reshape(B, EC), n - 1).astype(jnp.int32)
    return _build(B, EC, D)(score2, idx2, disp2, weight)
